# baseline (speedup 1.0000x reference)
import jax
import jax.numpy as jnp
from jax import lax
from jax.experimental import pallas as pl
from jax.experimental.pallas import tpu as pltpu
from jax.experimental.pallas import tpu_sc as plsc

B = 4
D = 3
ROW = 16
NW = 32

N_EDGES = 3200000
N_NODES = 100000

CH = 800
NK = 125

CHN = 800
NCHN = 125

_CP = pltpu.CompilerParams(needs_layout_passes=False, use_tc_tiling_on_sc=False)


def _wid():
    return lax.axis_index("s") * 2 + lax.axis_index("c")


def _rotate_body(t128_hbm, table_hbm, tbuf, obuf):
    wid = _wid()

    def chunk_body(k, carry):
        chunk = k * NW + wid

        @pl.when(chunk < NCHN)
        def _():
            base = chunk * CHN
            pltpu.sync_copy(
                t128_hbm.at[pl.ds(base, CHN), pl.ds(0, ROW)], tbuf)

            def group_body(g, carry2):
                ridx = g * 16 + lax.iota(jnp.int32, 16)
                for i in range(16):
                    r = g * 16 + i
                    v = tbuf[r, :]
                    plsc.store_scatter(
                        obuf,
                        [jnp.full((16,), r, jnp.int32),
                         jnp.bitwise_and(lax.iota(jnp.int32, 16) + r, 15)],
                        v)
                return carry2

            lax.fori_loop(0, CHN // 16, group_body, 0, unroll=False)
            pltpu.sync_copy(obuf, table_hbm.at[pl.ds(base, CHN)])
        return carry

    lax.fori_loop(0, (NCHN + NW - 1) // NW, chunk_body, 0, unroll=False)


def _compute_groups(srows, drows, sidxr, didxr, oacc):
    def group_body(g, carry):
        row0 = g * 16
        ridx = row0 + lax.iota(jnp.int32, 16)
        ns = jnp.bitwise_and(sidxr[pl.ds(row0, 16)], 15)
        nd = jnp.bitwise_and(didxr[pl.ds(row0, 16)], 15)

        def col(refr, rot, c):
            return plsc.load_gather(
                refr, [ridx, jnp.bitwise_and(rot + c, 15)])

        ev0 = col(srows, ns, 0) - col(drows, nd, 0)
        ev1 = col(srows, ns, 1) - col(drows, nd, 1)
        ev2 = col(srows, ns, 2) - col(drows, nd, 2)
        for b in range(B):
            acc = ((col(srows, ns, 3 + 3 * b) + col(drows, nd, 3 + 3 * b)) * ev0
                   + (col(srows, ns, 4 + 3 * b) + col(drows, nd, 4 + 3 * b)) * ev1
                   + (col(srows, ns, 5 + 3 * b) + col(drows, nd, 5 + 3 * b)) * ev2)
            oacc[b, pl.ds(row0, 16)] = acc * 0.5
        return carry

    lax.fori_loop(0, CH // 16, group_body, 0, unroll=False)


def _flux_body(table_hbm, src_hbm, dst_hbm, out_hbm,
               sidx0, sidx1, sidx2, sidx3, didx0, didx1, didx2, didx3,
               srows0, srows1, drows0, drows1, oacc0, oacc1,
               semI0, semI1, semI2, semI3, semG0, semG1, semO0, semO1):
    wid = _wid()
    sidx = [sidx0, sidx1, sidx2, sidx3]
    didx = [didx0, didx1, didx2, didx3]
    srows = [srows0, srows1]
    drows = [drows0, drows1]
    oacc = [oacc0, oacc1]
    semI = [semI0, semI1, semI2, semI3]
    semG = [semG0, semG1]
    semO = [semO0, semO1]

    def base_of(k):
        return (k * NW + wid) * CH

    def idx_copies(k, q):
        b = base_of(k)
        return (pltpu.make_async_copy(src_hbm.at[pl.ds(b, CH)], sidx[q], semI[q]),
                pltpu.make_async_copy(dst_hbm.at[pl.ds(b, CH)], didx[q], semI[q]))

    def gather_copies(s, q):
        cps = []
        for j in range(CH // 80):
            sl = pl.ds(j * 80, 80)
            cps.append(pltpu.make_async_copy(
                table_hbm.at[sidx[q].at[sl]], srows[s].at[sl], semG[s]))
            cps.append(pltpu.make_async_copy(
                table_hbm.at[didx[q].at[sl]], drows[s].at[sl], semG[s]))
        return cps

    def out_copies(k, s):
        base = base_of(k)
        return [pltpu.make_async_copy(
            oacc[s].at[b], out_hbm.at[pl.ds(b * N_EDGES + base, CH)], semO[s])
            for b in range(B)]

    def fire(copies):
        for cp in copies:
            cp.start()

    def drain(copies):
        for cp in copies:
            cp.wait()

    fire(idx_copies(0, 0))
    fire(idx_copies(1, 1))
    fire(idx_copies(2, 2))
    drain(idx_copies(0, 0))
    fire(gather_copies(0, 0))

    def quad_body(kk, carry):
        for s in range(4):
            k = kk * 4 + s
            rs = s % 2

            @pl.when(k < NK)
            def _():
                @pl.when(k + 1 < NK)
                def _():
                    drain(idx_copies(k + 1, (s + 1) % 4))
                    fire(gather_copies(rs ^ 1, (s + 1) % 4))

                @pl.when(k + 3 < NK)
                def _():
                    fire(idx_copies(k + 3, (s + 3) % 4))

                drain(gather_copies(rs, s))

                @pl.when(k >= 2)
                def _():
                    drain(out_copies(k - 2, rs))

                _compute_groups(srows[rs], drows[rs],
                                sidx[s], didx[s], oacc[rs])
                fire(out_copies(k, rs))
        return carry

    lax.fori_loop(0, (NK + 3) // 4, quad_body, 0, unroll=False)
    drain(out_copies(NK - 2, (NK - 2) % 2))
    drain(out_copies(NK - 1, (NK - 1) % 2))


_RBLK = 25600


def _relayout_body(f0, f1, f2, f3, out_ref):
    for b, f in enumerate((f0, f1, f2, f3)):
        out_ref[b, :] = f[...]


def _relayout(flat):
    grid = N_EDGES // _RBLK
    in_specs = [
        pl.BlockSpec((_RBLK,), lambda j, b=b: (b * grid + j,))
        for b in range(B)
    ]
    return pl.pallas_call(
        _relayout_body,
        grid=(grid,),
        in_specs=in_specs,
        out_specs=pl.BlockSpec((B, _RBLK), lambda j: (0, j)),
        out_shape=jax.ShapeDtypeStruct((B, N_EDGES), jnp.float32),
    )(flat, flat, flat, flat)


import numpy as _np

_WP = _np.zeros((D, 128), _np.float32)
for _d in range(D):
    _WP[_d, _d] = 1.0
_WN = [_np.zeros((D, 128), _np.float32) for _ in range(B)]
for _b in range(B):
    for _d in range(D):
        _WN[_b][_d, 3 + 3 * _b + _d] = 1.0


def kernel(node_vectors, points, edge_src, edge_dst):
    mesh = plsc.VectorSubcoreMesh(core_axis_name="c", subcore_axis_name="s",
                                  num_cores=2, num_subcores=16)

    t128 = jnp.dot(points, jnp.asarray(_WP), precision="highest")
    for b in range(B):
        t128 = t128 + jnp.dot(node_vectors[b], jnp.asarray(_WN[b]),
                              precision="highest")

    table = pl.kernel(
        _rotate_body,
        out_type=jax.ShapeDtypeStruct((N_NODES, ROW), jnp.float32),
        mesh=mesh,
        compiler_params=_CP,
        scratch_types=[
            pltpu.VMEM((CHN, ROW), jnp.float32),
            pltpu.VMEM((CHN, ROW), jnp.float32),
        ],
    )(t128)

    flat = pl.kernel(
        _flux_body,
        out_type=jax.ShapeDtypeStruct((B * N_EDGES,), jnp.float32),
        mesh=mesh,
        compiler_params=_CP,
        scratch_types=(
            [pltpu.VMEM((CH,), jnp.int32) for _ in range(8)]
            + [pltpu.VMEM((CH, ROW), jnp.float32) for _ in range(4)]
            + [pltpu.VMEM((B, CH), jnp.float32) for _ in range(2)]
            + [pltpu.SemaphoreType.DMA for _ in range(8)]
        ),
    )(table, edge_src, edge_dst)
    return _relayout(flat)


# speedup vs baseline: 1.1189x; 1.1189x over previous
import jax
import jax.numpy as jnp
from jax import lax
from jax.experimental import pallas as pl
from jax.experimental.pallas import tpu as pltpu
from jax.experimental.pallas import tpu_sc as plsc

B = 4
D = 3
ROW = 16
NW = 32

N_EDGES = 3200000
N_NODES = 100000

CH = 800
NK = 125

CHN = 800
NCHN = 125

_CP = pltpu.CompilerParams(needs_layout_passes=False, use_tc_tiling_on_sc=False)


def _wid():
    return lax.axis_index("s") * 2 + lax.axis_index("c")


def _pack_body(nv_hbm, pts_hbm, table_hbm, pbuf, nvbuf, obuf):
    wid = _wid()

    def chunk_body(k, carry):
        chunk = k * NW + wid

        @pl.when(chunk < NCHN)
        def _():
            base = chunk * CHN
            pltpu.sync_copy(pts_hbm.at[pl.ds(base * D, CHN * D)], pbuf)
            for b in range(B):
                pltpu.sync_copy(
                    nv_hbm.at[pl.ds((b * N_NODES + base) * D, CHN * D)],
                    nvbuf.at[pl.ds(b * CHN * D, CHN * D)])

            def group_body(g, carry2):
                ridx = g * 16 + lax.iota(jnp.int32, 16)

                def put(col, v):
                    plsc.store_scatter(
                        obuf, [ridx, jnp.bitwise_and(ridx + col, 15)], v)

                for d in range(D):
                    put(d, plsc.load_gather(pbuf, [ridx * D + d]))
                for b in range(B):
                    for d in range(D):
                        v = plsc.load_gather(
                            nvbuf, [(b * CHN + ridx) * D + d])
                        put(3 + 3 * b + d, v)
                return carry2

            lax.fori_loop(0, CHN // 16, group_body, 0, unroll=False)
            pltpu.sync_copy(obuf, table_hbm.at[pl.ds(base, CHN)])
        return carry

    lax.fori_loop(0, (NCHN + NW - 1) // NW, chunk_body, 0, unroll=False)


def _compute_groups(srows, drows, sidxr, didxr, oacc):
    def group_body(g, carry):
        row0 = g * 16
        ridx = row0 + lax.iota(jnp.int32, 16)
        ns = jnp.bitwise_and(sidxr[pl.ds(row0, 16)], 15)
        nd = jnp.bitwise_and(didxr[pl.ds(row0, 16)], 15)

        def col(refr, rot, c):
            return plsc.load_gather(
                refr, [ridx, jnp.bitwise_and(rot + c, 15)])

        ev0 = col(srows, ns, 0) - col(drows, nd, 0)
        ev1 = col(srows, ns, 1) - col(drows, nd, 1)
        ev2 = col(srows, ns, 2) - col(drows, nd, 2)
        for b in range(B):
            acc = ((col(srows, ns, 3 + 3 * b) + col(drows, nd, 3 + 3 * b)) * ev0
                   + (col(srows, ns, 4 + 3 * b) + col(drows, nd, 4 + 3 * b)) * ev1
                   + (col(srows, ns, 5 + 3 * b) + col(drows, nd, 5 + 3 * b)) * ev2)
            oacc[b, pl.ds(row0, 16)] = acc * 0.5
        return carry

    lax.fori_loop(0, CH // 16, group_body, 0, unroll=False)


def _flux_body(table_hbm, src_hbm, dst_hbm, out_hbm,
               sidx0, sidx1, sidx2, didx0, didx1, didx2,
               srows0, srows1, srows2, drows0, drows1, drows2,
               oacc0, oacc1, oacc2,
               semI0, semI1, semI2, semG0, semG1, semG2,
               semO0, semO1, semO2):
    wid = _wid()
    sidx = [sidx0, sidx1, sidx2]
    didx = [didx0, didx1, didx2]
    srows = [srows0, srows1, srows2]
    drows = [drows0, drows1, drows2]
    oacc = [oacc0, oacc1, oacc2]
    semI = [semI0, semI1, semI2]
    semG = [semG0, semG1, semG2]
    semO = [semO0, semO1, semO2]

    def base_of(k):
        return (k * NW + wid) * CH

    def idx_copies(k, q):
        b = base_of(k)
        return (pltpu.make_async_copy(src_hbm.at[pl.ds(b, CH)], sidx[q], semI[q]),
                pltpu.make_async_copy(dst_hbm.at[pl.ds(b, CH)], didx[q], semI[q]))

    def gather_copies(s, q):
        cps = []
        for j in range(CH // 80):
            sl = pl.ds(j * 80, 80)
            cps.append(pltpu.make_async_copy(
                table_hbm.at[sidx[q].at[sl]], srows[s].at[sl], semG[s]))
            cps.append(pltpu.make_async_copy(
                table_hbm.at[didx[q].at[sl]], drows[s].at[sl], semG[s]))
        return cps

    def out_copies(k, s):
        base = base_of(k)
        return [pltpu.make_async_copy(
            oacc[s].at[b], out_hbm.at[pl.ds(b * N_EDGES + base, CH)], semO[s])
            for b in range(B)]

    def fire(copies):
        for cp in copies:
            cp.start()

    def drain(copies):
        for cp in copies:
            cp.wait()

    fire(idx_copies(0, 0))
    fire(idx_copies(1, 1))
    fire(idx_copies(2, 2))
    drain(idx_copies(0, 0))
    fire(gather_copies(0, 0))
    drain(idx_copies(1, 1))
    fire(gather_copies(1, 1))

    def tri_body(kk, carry):
        for s in range(3):
            k = kk * 3 + s

            @pl.when(k < NK)
            def _():
                drain(gather_copies(s, s))
                @pl.when(k + 2 < NK)
                def _():
                    drain(idx_copies(k + 2, (s + 2) % 3))
                    fire(gather_copies((s + 2) % 3, (s + 2) % 3))

                @pl.when(k >= 3)
                def _():
                    drain(out_copies(k - 3, s))

                _compute_groups(srows[s], drows[s],
                                sidx[s], didx[s], oacc[s])
                @pl.when(k + 3 < NK)
                def _():
                    fire(idx_copies(k + 3, s))

                fire(out_copies(k, s))
        return carry

    lax.fori_loop(0, (NK + 2) // 3, tri_body, 0, unroll=False)
    for k in (NK - 3, NK - 2, NK - 1):
        drain(out_copies(k, k % 3))


_RBLK = 128000


def _relayout_body(f0, f1, f2, f3, out_ref):
    for b, f in enumerate((f0, f1, f2, f3)):
        out_ref[b, :] = f[...]


def _relayout(flat):
    grid = N_EDGES // _RBLK
    in_specs = [
        pl.BlockSpec((_RBLK,), lambda j, b=b: (b * grid + j,))
        for b in range(B)
    ]
    return pl.pallas_call(
        _relayout_body,
        grid=(grid,),
        in_specs=in_specs,
        out_specs=pl.BlockSpec((B, _RBLK), lambda j: (0, j)),
        out_shape=jax.ShapeDtypeStruct((B, N_EDGES), jnp.float32),
    )(flat, flat, flat, flat)


def kernel(node_vectors, points, edge_src, edge_dst):
    mesh = plsc.VectorSubcoreMesh(core_axis_name="c", subcore_axis_name="s",
                                  num_cores=2, num_subcores=16)

    table = pl.kernel(
        _pack_body,
        out_type=jax.ShapeDtypeStruct((N_NODES, ROW), jnp.float32),
        mesh=mesh,
        compiler_params=_CP,
        scratch_types=[
            pltpu.VMEM((CHN * D,), jnp.float32),
            pltpu.VMEM((B * CHN * D,), jnp.float32),
            pltpu.VMEM((CHN, ROW), jnp.float32),
        ],
    )(node_vectors.reshape(-1), points.reshape(-1))

    flat = pl.kernel(
        _flux_body,
        out_type=jax.ShapeDtypeStruct((B * N_EDGES,), jnp.float32),
        mesh=mesh,
        compiler_params=_CP,
        scratch_types=(
            [pltpu.VMEM((CH,), jnp.int32) for _ in range(6)]
            + [pltpu.VMEM((CH, ROW), jnp.float32) for _ in range(6)]
            + [pltpu.VMEM((B, CH), jnp.float32) for _ in range(3)]
            + [pltpu.SemaphoreType.DMA for _ in range(9)]
        ),
    )(table, edge_src, edge_dst)
    return _relayout(flat)


# speedup vs baseline: 1.2264x; 1.0961x over previous
import jax
import jax.numpy as jnp
from jax import lax
from jax.experimental import pallas as pl
from jax.experimental.pallas import tpu as pltpu
from jax.experimental.pallas import tpu_sc as plsc

B = 4
D = 3
ROW = 16
NW = 32

N_EDGES = 3200000
N_NODES = 100000

CH = 800
NK = 125

CHN = 800
NCHN = 125

_CP = pltpu.CompilerParams(needs_layout_passes=False, use_tc_tiling_on_sc=False)


def _wid():
    return lax.axis_index("s") * 2 + lax.axis_index("c")


def _pack_body(nv_hbm, pts_hbm, table_hbm, pbuf, nvbuf, obuf):
    wid = _wid()

    def chunk_body(k, carry):
        chunk = k * NW + wid

        @pl.when(chunk < NCHN)
        def _():
            base = chunk * CHN
            pltpu.sync_copy(pts_hbm.at[pl.ds(base * D, CHN * D)], pbuf)
            for b in range(B):
                pltpu.sync_copy(
                    nv_hbm.at[pl.ds((b * N_NODES + base) * D, CHN * D)],
                    nvbuf.at[pl.ds(b * CHN * D, CHN * D)])

            def group_body(g, carry2):
                ridx = g * 16 + lax.iota(jnp.int32, 16)

                def put(col, v):
                    plsc.store_scatter(
                        obuf, [ridx, jnp.bitwise_and(ridx + col, 15)], v)

                for d in range(D):
                    put(d, plsc.load_gather(pbuf, [ridx * D + d]))
                for b in range(B):
                    for d in range(D):
                        v = plsc.load_gather(
                            nvbuf, [(b * CHN + ridx) * D + d])
                        put(3 + 3 * b + d, v)
                return carry2

            lax.fori_loop(0, CHN // 16, group_body, 0, unroll=False)
            pltpu.sync_copy(obuf, table_hbm.at[pl.ds(base, CHN)])
        return carry

    lax.fori_loop(0, (NCHN + NW - 1) // NW, chunk_body, 0, unroll=False)


def _compute_groups(srows, drows, sidxr, didxr, oacc):
    def group_body(g, carry):
        row0 = g * 16
        ridx = row0 + lax.iota(jnp.int32, 16)
        ns = jnp.bitwise_and(sidxr[pl.ds(row0, 16)], 15)
        nd = jnp.bitwise_and(didxr[pl.ds(row0, 16)], 15)

        def col(refr, rot, c):
            return plsc.load_gather(
                refr, [ridx, jnp.bitwise_and(rot + c, 15)])

        ev0 = col(srows, ns, 0) - col(drows, nd, 0)
        ev1 = col(srows, ns, 1) - col(drows, nd, 1)
        ev2 = col(srows, ns, 2) - col(drows, nd, 2)
        for b in range(B):
            acc = ((col(srows, ns, 3 + 3 * b) + col(drows, nd, 3 + 3 * b)) * ev0
                   + (col(srows, ns, 4 + 3 * b) + col(drows, nd, 4 + 3 * b)) * ev1
                   + (col(srows, ns, 5 + 3 * b) + col(drows, nd, 5 + 3 * b)) * ev2)
            oacc[b, pl.ds(row0, 16)] = acc * 0.5
        return carry

    lax.fori_loop(0, CH // 16, group_body, 0, unroll=False)


def _flux_body(table_hbm, src_hbm, dst_hbm, out_hbm,
               sidx0, sidx1, sidx2, sidx3, didx0, didx1, didx2, didx3,
               srows0, srows1, drows0, drows1, oacc0, oacc1,
               semI0, semI1, semI2, semI3, semG0, semG1, semO0, semO1):
    wid = _wid()
    sidx = [sidx0, sidx1, sidx2, sidx3]
    didx = [didx0, didx1, didx2, didx3]
    srows = [srows0, srows1]
    drows = [drows0, drows1]
    oacc = [oacc0, oacc1]
    semI = [semI0, semI1, semI2, semI3]
    semG = [semG0, semG1]
    semO = [semO0, semO1]

    def base_of(k):
        return (k * NW + wid) * CH

    def idx_copies(k, q):
        b = base_of(k)
        return (pltpu.make_async_copy(src_hbm.at[pl.ds(b, CH)], sidx[q], semI[q]),
                pltpu.make_async_copy(dst_hbm.at[pl.ds(b, CH)], didx[q], semI[q]))

    def gather_copies(s, q):
        cps = []
        for j in range(CH // 80):
            sl = pl.ds(j * 80, 80)
            cps.append(pltpu.make_async_copy(
                table_hbm.at[sidx[q].at[sl]], srows[s].at[sl], semG[s]))
            cps.append(pltpu.make_async_copy(
                table_hbm.at[didx[q].at[sl]], drows[s].at[sl], semG[s]))
        return cps

    def out_copies(k, s):
        base = base_of(k)
        return [pltpu.make_async_copy(
            oacc[s].at[b], out_hbm.at[pl.ds(b * N_EDGES + base, CH)], semO[s])
            for b in range(B)]

    def fire(copies):
        for cp in copies:
            cp.start()

    def drain(copies):
        for cp in copies:
            cp.wait()

    fire(idx_copies(0, 0))
    fire(idx_copies(1, 1))
    fire(idx_copies(2, 2))
    drain(idx_copies(0, 0))
    fire(gather_copies(0, 0))

    def quad_body(kk, carry):
        for s in range(4):
            k = kk * 4 + s
            rs = s % 2

            @pl.when(k < NK)
            def _():
                @pl.when(k + 1 < NK)
                def _():
                    drain(idx_copies(k + 1, (s + 1) % 4))
                    fire(gather_copies(rs ^ 1, (s + 1) % 4))

                @pl.when(k + 3 < NK)
                def _():
                    fire(idx_copies(k + 3, (s + 3) % 4))

                drain(gather_copies(rs, s))

                @pl.when(k >= 2)
                def _():
                    drain(out_copies(k - 2, rs))

                _compute_groups(srows[rs], drows[rs],
                                sidx[s], didx[s], oacc[rs])
                fire(out_copies(k, rs))
        return carry

    lax.fori_loop(0, (NK + 3) // 4, quad_body, 0, unroll=False)
    drain(out_copies(NK - 2, (NK - 2) % 2))
    drain(out_copies(NK - 1, (NK - 1) % 2))


_RBLK = 128000


def _relayout_body(f0, f1, f2, f3, out_ref):
    for b, f in enumerate((f0, f1, f2, f3)):
        out_ref[b, :] = f[...]


def _relayout(flat):
    grid = N_EDGES // _RBLK
    in_specs = [
        pl.BlockSpec((_RBLK,), lambda j, b=b: (b * grid + j,))
        for b in range(B)
    ]
    return pl.pallas_call(
        _relayout_body,
        grid=(grid,),
        in_specs=in_specs,
        out_specs=pl.BlockSpec((B, _RBLK), lambda j: (0, j)),
        out_shape=jax.ShapeDtypeStruct((B, N_EDGES), jnp.float32),
    )(flat, flat, flat, flat)


def kernel(node_vectors, points, edge_src, edge_dst):
    mesh = plsc.VectorSubcoreMesh(core_axis_name="c", subcore_axis_name="s",
                                  num_cores=2, num_subcores=16)

    table = pl.kernel(
        _pack_body,
        out_type=jax.ShapeDtypeStruct((N_NODES, ROW), jnp.float32),
        mesh=mesh,
        compiler_params=_CP,
        scratch_types=[
            pltpu.VMEM((CHN * D,), jnp.float32),
            pltpu.VMEM((B * CHN * D,), jnp.float32),
            pltpu.VMEM((CHN, ROW), jnp.float32),
        ],
    )(node_vectors.reshape(-1), points.reshape(-1))

    flat = pl.kernel(
        _flux_body,
        out_type=jax.ShapeDtypeStruct((B * N_EDGES,), jnp.float32),
        mesh=mesh,
        compiler_params=_CP,
        scratch_types=(
            [pltpu.VMEM((CH,), jnp.int32) for _ in range(8)]
            + [pltpu.VMEM((CH, ROW), jnp.float32) for _ in range(4)]
            + [pltpu.VMEM((B, CH), jnp.float32) for _ in range(2)]
            + [pltpu.SemaphoreType.DMA for _ in range(8)]
        ),
    )(table, edge_src, edge_dst)
    return _relayout(flat)
